# baseline (speedup 1.0000x reference)
import functools

import jax
import jax.numpy as jnp
import numpy as np
from jax import lax
from jax.experimental import pallas as pl
from jax.experimental.pallas import tpu as pltpu
from jax.experimental.pallas import tpu_sc as plsc

H = 128
PD = 27
LD = 13
KNN = 16
NG = 16
LAYERS = 3
B = 256
NP_ = 8000
NL = 2000
N = NP_ + NL
NPAD = 10240
NEPAD = NPAD * KNN
KROWS = 128
KGRID = NPAD // KROWS
MROWS = 256
MEDGE = MROWS * KNN
MGRID = NPAD // MROWS

_GAMMA = 1.0 / (2.0 * (30.0 / NG) ** 2)


def _embed_body(fp_ref, wp_ref, bp_ref, fl_ref, wl_ref, bl_ref, hp_ref, hl_ref):
    hp_ref[...] = jnp.dot(fp_ref[...], wp_ref[...],
                          preferred_element_type=jnp.float32) + bp_ref[...]
    hl_ref[...] = jnp.dot(fl_ref[...], wl_ref[...],
                          preferred_element_type=jnp.float32) + bl_ref[...]


def _embed(feat_p, Wp, bp, feat_l, Wl, bl):
    return pl.pallas_call(
        _embed_body,
        out_shape=(jax.ShapeDtypeStruct((NP_, H), jnp.float32),
                   jax.ShapeDtypeStruct((NL, H), jnp.float32)),
    )(feat_p, Wp, bp.reshape(1, H), feat_l, Wl, bl.reshape(1, H))


WWIN = 512


def _knn_body(pos8_ref, posT_ref, sq_ref, brow_ref, bcol_ref, cnt_ref,
              idx_ref, d2_ref, sc_ref, sw_ref):
    pid = pl.program_id(0)
    pc = pos8_ref[...]
    rowsq = jnp.sum(pc * pc, axis=1, keepdims=True)
    grow = pid * KROWS + lax.broadcasted_iota(jnp.int32, (KROWS, 1), 0)
    bc = bcol_ref[...].reshape(KROWS, 1)
    br = brow_ref[...]

    bc_min = jnp.min(jnp.where(bc >= 0, bc, jnp.int32(2 ** 30)))
    bc_max = jnp.max(bc)
    col1 = lax.broadcasted_iota(jnp.int32, (1, NPAD), 1)
    lo = jnp.min(jnp.where(br == bc_min, col1, NPAD))
    hi = jnp.max(jnp.where(br == bc_max, col1 + 1, 0))
    lo_al = pl.multiple_of(jnp.clip((lo // 128) * 128, 0, NPAD - WWIN), 128)
    bidx = lax.broadcasted_iota(jnp.int32, (1, B), 1)
    inrange = (bidx >= bc_min) & (bidx <= bc_max)
    minseg = jnp.min(jnp.where(inrange, cnt_ref[...], jnp.int32(2 ** 30)))
    fast = (minseg >= KNN + 1) & (hi - lo_al <= WWIN)

    @pl.when(fast)
    def _fast():
        pcw = posT_ref[:, pl.ds(lo_al, WWIN)]
        d2 = rowsq + sq_ref[:, pl.ds(lo_al, WWIN)] - 2.0 * jnp.dot(
            pc, pcw, preferred_element_type=jnp.float32)
        colw = lo_al + lax.broadcasted_iota(jnp.int32, (KROWS, WWIN), 1)
        pen = jnp.where(brow_ref[:, pl.ds(lo_al, WWIN)] != bc, 1e9, 0.0)
        pen = pen + jnp.where(colw == grow, 1e9, 0.0)
        sw_ref[...] = d2 + pen
        for j in range(KNN):
            s = sw_ref[...]
            m = jnp.min(s, axis=1, keepdims=True)
            idx = jnp.min(jnp.where(s == m, colw, NPAD), axis=1, keepdims=True)
            idx_ref[:, j:j + 1] = idx
            d2_ref[:, j:j + 1] = m
            sw_ref[...] = jnp.where(colw == idx, jnp.inf, s)

    @pl.when(jnp.logical_not(fast))
    def _slow():
        dot = jnp.dot(pc, posT_ref[...], preferred_element_type=jnp.float32)
        d2 = rowsq + sq_ref[...] - 2.0 * dot
        col = lax.broadcasted_iota(jnp.int32, (KROWS, NPAD), 1)
        pen = jnp.where(br != bc, 1e9, 0.0)
        pen = pen + jnp.where(col == grow, 1e9, 0.0)
        sc_ref[...] = d2 + pen
        for j in range(KNN):
            s = sc_ref[...]
            m = jnp.min(s, axis=1, keepdims=True)
            idx = jnp.min(jnp.where(s == m, col, NPAD), axis=1, keepdims=True)
            idx_ref[:, j:j + 1] = idx
            psel = jnp.where(m < 5e8, 0.0, jnp.where(m < 1.5e9, 1e9, 2e9))
            d2_ref[:, j:j + 1] = m - psel
            sc_ref[...] = jnp.where(col == idx, jnp.inf, s)


def _knn(pos8, posT, sq_row, batch_row, batch_col, counts):
    return pl.pallas_call(
        _knn_body,
        grid=(KGRID,),
        in_specs=[
            pl.BlockSpec((KROWS, 8), lambda i: (i, 0)),
            pl.BlockSpec((8, NPAD), lambda i: (0, 0)),
            pl.BlockSpec((1, NPAD), lambda i: (0, 0)),
            pl.BlockSpec((1, NPAD), lambda i: (0, 0)),
            pl.BlockSpec((KROWS, 1), lambda i: (i, 0)),
            pl.BlockSpec((1, B), lambda i: (0, 0)),
        ],
        out_specs=(pl.BlockSpec((KROWS, KNN), lambda i: (i, 0)),
                   pl.BlockSpec((KROWS, KNN), lambda i: (i, 0))),
        out_shape=(jax.ShapeDtypeStruct((NPAD, KNN), jnp.int32),
                   jax.ShapeDtypeStruct((NPAD, KNN), jnp.float32)),
        scratch_shapes=[pltpu.VMEM((KROWS, NPAD), jnp.float32),
                        pltpu.VMEM((KROWS, WWIN), jnp.float32)],
    )(pos8, posT, sq_row, batch_row, batch_col, counts)


def _sc_gather(idx, table):
    D = table.shape[1]
    nep = idx.shape[0]
    info = plsc.get_sparse_core_info()
    nc, ns = info.num_cores, info.num_subcores
    nw = nc * ns
    per_w = nep // nw
    nch = per_w // KROWS

    nring = 4
    nrounds = nch // nring

    @functools.partial(
        pl.kernel,
        mesh=plsc.VectorSubcoreMesh(core_axis_name="c", subcore_axis_name="s"),
        out_type=jax.ShapeDtypeStruct((nep, D), jnp.float32),
        scratch_types=[
            pltpu.VMEM((nring, KROWS), jnp.int32),
            pltpu.VMEM((nring, KROWS, D), jnp.float32),
            pltpu.SemaphoreType.DMA,
            pltpu.SemaphoreType.DMA,
            pltpu.SemaphoreType.DMA,
        ],
    )
    def gk(idx_hbm, tab_hbm, out_hbm, idx_v, rows_v, isem, gsem, wsem):
        wid = lax.axis_index("s") * nc + lax.axis_index("c")
        base = wid * per_w

        def round_body(r, carry):
            offs = [base + (r * nring + b) * KROWS for b in range(nring)]
            hi = [pltpu.async_copy(idx_hbm.at[pl.ds(offs[b], KROWS)],
                                   idx_v.at[b], isem) for b in range(nring)]
            hg = []
            for b in range(nring):
                hi[b].wait()
                hg.append(pltpu.async_copy(tab_hbm.at[idx_v.at[b]],
                                           rows_v.at[b], gsem))
            hw = []
            for b in range(nring):
                hg[b].wait()
                hw.append(pltpu.async_copy(rows_v.at[b],
                                           out_hbm.at[pl.ds(offs[b], KROWS)],
                                           wsem))
            for b in range(nring):
                hw[b].wait()
            return carry

        lax.fori_loop(0, nrounds, round_body, 0)

    return gk(idx, table)


def _mp_body(h_ref, ysrc_ref, d2_ref, w2_ref, bm_ref, wu_ref,
             bu_ref, wn_ref, bn_ref, hout_ref, ynext_ref):
    d2 = jnp.maximum(d2_ref[...], 0.0)
    d3 = jnp.sqrt(d2 + 1e-8).reshape(MROWS, KNN, 1)
    centers = lax.broadcasted_iota(jnp.int32, (1, 1, NG), 2).astype(jnp.float32) * 2.0
    rbf = jnp.exp(-_GAMMA * (d3 - centers) ** 2).reshape(MEDGE, NG)
    rw = jnp.dot(rbf, w2_ref[...], preferred_element_type=jnp.float32)
    m = jnp.maximum(ysrc_ref[...] + rw + bm_ref[...], 0.0)
    agg = jnp.sum(m.reshape(MROWS, KNN, H), axis=1)
    upd = jnp.maximum(
        jnp.dot(agg, wu_ref[...], preferred_element_type=jnp.float32)
        + bu_ref[...], 0.0)
    hout = h_ref[...] + upd
    hout_ref[...] = hout
    ynext_ref[...] = jnp.dot(hout, wn_ref[...],
                             preferred_element_type=jnp.float32) + bn_ref[...]


def _mp_layer_half(h, y_src_half, d2sel, W2, bm, Wu, bu, Wn, bn, half):
    hb = (MGRID // 2) * half
    return pl.pallas_call(
        _mp_body,
        grid=(MGRID // 2,),
        in_specs=[
            pl.BlockSpec((MROWS, H), lambda i: (i + hb, 0)),
            pl.BlockSpec((MEDGE, H), lambda i: (i, 0)),
            pl.BlockSpec((MROWS, KNN), lambda i: (i + hb, 0)),
            pl.BlockSpec((NG, H), lambda i: (0, 0)),
            pl.BlockSpec((1, H), lambda i: (0, 0)),
            pl.BlockSpec((H, H), lambda i: (0, 0)),
            pl.BlockSpec((1, H), lambda i: (0, 0)),
            pl.BlockSpec((H, H), lambda i: (0, 0)),
            pl.BlockSpec((1, H), lambda i: (0, 0)),
        ],
        out_specs=(pl.BlockSpec((MROWS, H), lambda i: (i, 0)),
                   pl.BlockSpec((MROWS, H), lambda i: (i, 0))),
        out_shape=(jax.ShapeDtypeStruct((NPAD // 2, H), jnp.float32),
                   jax.ShapeDtypeStruct((NPAD // 2, H), jnp.float32)),
    )(h, y_src_half, d2sel, W2, bm, Wu, bu, Wn, bn)


def _y0_body(h_ref, w_ref, y_ref):
    y_ref[...] = jnp.dot(h_ref[...], w_ref[...],
                         preferred_element_type=jnp.float32)


def _y0(h, W1):
    return pl.pallas_call(
        _y0_body,
        grid=(MGRID,),
        in_specs=[pl.BlockSpec((MROWS, H), lambda i: (i, 0)),
                  pl.BlockSpec((H, H), lambda i: (0, 0))],
        out_specs=pl.BlockSpec((MROWS, H), lambda i: (i, 0)),
        out_shape=jax.ShapeDtypeStruct((NPAD, H), jnp.float32),
    )(h, W1)


def kernel(protein_pos, protein_atom_feature, ligand_pos, ligand_atom_feature,
           batch_protein, batch_ligand, Wp, bp, Wl, bl, W_msg, b_msg,
           W_upd, b_upd, Wf, bf):
    bpn = batch_protein.astype(jnp.int32)
    bln = batch_ligand.astype(jnp.int32)

    h_p, h_l = _embed(protein_atom_feature, Wp, bp, ligand_atom_feature, Wl, bl)

    dest_p = jnp.arange(NP_, dtype=jnp.int32) + jnp.searchsorted(
        bln, bpn, side="left").astype(jnp.int32)
    dest_l = jnp.searchsorted(bpn, bln, side="right").astype(jnp.int32) + \
        jnp.arange(NL, dtype=jnp.int32)
    dest = jnp.concatenate([dest_p, dest_l], 0)

    pos_all = jnp.concatenate([protein_pos, ligand_pos], 0)
    h_all = jnp.concatenate([h_p, h_l], 0)
    batch_all = jnp.concatenate([bpn, bln], 0)
    mask_all = jnp.concatenate([jnp.ones((NP_,), bool), jnp.zeros((NL,), bool)], 0)

    pos_ctx = jnp.zeros((N, 3), jnp.float32).at[dest].set(pos_all)
    h_ctx = jnp.zeros((N, H), jnp.float32).at[dest].set(h_all)
    batch_ctx = jnp.zeros((N,), jnp.int32).at[dest].set(batch_all)
    protein_mask = jnp.zeros((N,), bool).at[dest].set(mask_all)

    pos16 = jnp.zeros((NPAD, 16), jnp.float32).at[:N, :3].set(pos_ctx)
    pos8 = pos16[:, :8]
    posT = pos8.T.reshape(8, NPAD)
    sq = jnp.sum(pos_ctx * pos_ctx, axis=1)
    sq_row = jnp.full((1, NPAD), 3e9, jnp.float32).at[0, :N].set(sq)
    bpad = jnp.full((NPAD,), -1, jnp.int32).at[:N].set(batch_ctx)
    batch_row = bpad.reshape(1, NPAD)
    batch_col = bpad.reshape(NPAD, 1)
    hpad = jnp.zeros((NPAD, H), jnp.float32).at[:N].set(h_ctx)

    counts = jnp.zeros((B,), jnp.int32).at[batch_ctx].add(1).reshape(1, B)
    knn_idx, d2sel = _knn(pos8, posT, sq_row, batch_row, batch_col, counts)
    src = knn_idx.reshape(-1)

    h = hpad
    y = _y0(h, W_msg[0, :H, :])
    Wf_pad = jnp.zeros((H, H), jnp.float32).at[:, :1].set(Wf)
    bf_pad = jnp.zeros((1, H), jnp.float32).at[0, :1].set(bf)
    zrow = jnp.zeros((1, H), jnp.float32)
    src_a, src_b = src[:NEPAD // 2], src[NEPAD // 2:]
    for l in range(LAYERS):
        if l + 1 < LAYERS:
            Wn, bn = W_msg[l + 1, :H, :], zrow
        else:
            Wn, bn = Wf_pad, bf_pad
        wargs = (W_msg[l, H:, :], b_msg[l].reshape(1, H),
                 W_upd[l], b_upd[l].reshape(1, H), Wn, bn)
        ya = _sc_gather(src_a, y)
        yb = _sc_gather(src_b, y)
        ha, yna = _mp_layer_half(h, ya, d2sel, *wargs, half=0)
        hb, ynb = _mp_layer_half(h, yb, d2sel, *wargs, half=1)
        h = jnp.concatenate([ha, hb], 0)
        y = jnp.concatenate([yna, ynb], 0)

    focal_pred = y[:N, :1]
    return focal_pred, protein_mask, h[:N]


# speedup vs baseline: 1.0119x; 1.0119x over previous
import functools

import jax
import jax.numpy as jnp
import numpy as np
from jax import lax
from jax.experimental import pallas as pl
from jax.experimental.pallas import tpu as pltpu
from jax.experimental.pallas import tpu_sc as plsc

H = 128
PD = 27
LD = 13
KNN = 16
NG = 16
LAYERS = 3
B = 256
NP_ = 8000
NL = 2000
N = NP_ + NL
NPAD = 10240
NEPAD = NPAD * KNN
KROWS = 128
KGRID = NPAD // KROWS
MROWS = 256
MEDGE = MROWS * KNN
MGRID = NPAD // MROWS

_GAMMA = 1.0 / (2.0 * (30.0 / NG) ** 2)


def _embed_body(fp_ref, wp_ref, bp_ref, fl_ref, wl_ref, bl_ref, hp_ref, hl_ref):
    hp_ref[...] = jnp.dot(fp_ref[...], wp_ref[...],
                          preferred_element_type=jnp.float32) + bp_ref[...]
    hl_ref[...] = jnp.dot(fl_ref[...], wl_ref[...],
                          preferred_element_type=jnp.float32) + bl_ref[...]


def _embed(feat_p, Wp, bp, feat_l, Wl, bl):
    return pl.pallas_call(
        _embed_body,
        out_shape=(jax.ShapeDtypeStruct((NP_, H), jnp.float32),
                   jax.ShapeDtypeStruct((NL, H), jnp.float32)),
    )(feat_p, Wp, bp.reshape(1, H), feat_l, Wl, bl.reshape(1, H))


WWIN = 512


def _knn_body(pos8_ref, posT_ref, sq_ref, brow_ref, bcol_ref, cnt_ref,
              idx_ref, d2_ref, sc_ref, sw_ref):
    pid = pl.program_id(0)
    pc = pos8_ref[...]
    rowsq = jnp.sum(pc * pc, axis=1, keepdims=True)
    grow = pid * KROWS + lax.broadcasted_iota(jnp.int32, (KROWS, 1), 0)
    bc = bcol_ref[...].reshape(KROWS, 1)
    br = brow_ref[...]

    bc_min = jnp.min(jnp.where(bc >= 0, bc, jnp.int32(2 ** 30)))
    bc_max = jnp.max(bc)
    col1 = lax.broadcasted_iota(jnp.int32, (1, NPAD), 1)
    lo = jnp.min(jnp.where(br == bc_min, col1, NPAD))
    hi = jnp.max(jnp.where(br == bc_max, col1 + 1, 0))
    lo_al = pl.multiple_of(jnp.clip((lo // 128) * 128, 0, NPAD - WWIN), 128)
    bidx = lax.broadcasted_iota(jnp.int32, (1, B), 1)
    inrange = (bidx >= bc_min) & (bidx <= bc_max)
    minseg = jnp.min(jnp.where(inrange, cnt_ref[...], jnp.int32(2 ** 30)))
    fast = (minseg >= KNN + 1) & (hi - lo_al <= WWIN)

    @pl.when(fast)
    def _fast():
        pcw = posT_ref[:, pl.ds(lo_al, WWIN)]
        d2 = rowsq + sq_ref[:, pl.ds(lo_al, WWIN)] - 2.0 * jnp.dot(
            pc, pcw, preferred_element_type=jnp.float32)
        colw = lo_al + lax.broadcasted_iota(jnp.int32, (KROWS, WWIN), 1)
        pen = jnp.where(brow_ref[:, pl.ds(lo_al, WWIN)] != bc, 1e9, 0.0)
        pen = pen + jnp.where(colw == grow, 1e9, 0.0)
        sw_ref[...] = d2 + pen
        for j in range(KNN):
            s = sw_ref[...]
            m = jnp.min(s, axis=1, keepdims=True)
            idx = jnp.min(jnp.where(s == m, colw, NPAD), axis=1, keepdims=True)
            idx_ref[:, j:j + 1] = idx
            d2_ref[:, j:j + 1] = m
            sw_ref[...] = jnp.where(colw == idx, jnp.inf, s)

    @pl.when(jnp.logical_not(fast))
    def _slow():
        dot = jnp.dot(pc, posT_ref[...], preferred_element_type=jnp.float32)
        d2 = rowsq + sq_ref[...] - 2.0 * dot
        col = lax.broadcasted_iota(jnp.int32, (KROWS, NPAD), 1)
        pen = jnp.where(br != bc, 1e9, 0.0)
        pen = pen + jnp.where(col == grow, 1e9, 0.0)
        sc_ref[...] = d2 + pen
        for j in range(KNN):
            s = sc_ref[...]
            m = jnp.min(s, axis=1, keepdims=True)
            idx = jnp.min(jnp.where(s == m, col, NPAD), axis=1, keepdims=True)
            idx_ref[:, j:j + 1] = idx
            psel = jnp.where(m < 5e8, 0.0, jnp.where(m < 1.5e9, 1e9, 2e9))
            d2_ref[:, j:j + 1] = m - psel
            sc_ref[...] = jnp.where(col == idx, jnp.inf, s)


def _knn(pos8, posT, sq_row, batch_row, batch_col, counts):
    return pl.pallas_call(
        _knn_body,
        grid=(KGRID,),
        in_specs=[
            pl.BlockSpec((KROWS, 8), lambda i: (i, 0)),
            pl.BlockSpec((8, NPAD), lambda i: (0, 0)),
            pl.BlockSpec((1, NPAD), lambda i: (0, 0)),
            pl.BlockSpec((1, NPAD), lambda i: (0, 0)),
            pl.BlockSpec((KROWS, 1), lambda i: (i, 0)),
            pl.BlockSpec((1, B), lambda i: (0, 0)),
        ],
        out_specs=(pl.BlockSpec((KROWS, KNN), lambda i: (i, 0)),
                   pl.BlockSpec((KROWS, KNN), lambda i: (i, 0))),
        out_shape=(jax.ShapeDtypeStruct((NPAD, KNN), jnp.int32),
                   jax.ShapeDtypeStruct((NPAD, KNN), jnp.float32)),
        scratch_shapes=[pltpu.VMEM((KROWS, NPAD), jnp.float32),
                        pltpu.VMEM((KROWS, WWIN), jnp.float32)],
    )(pos8, posT, sq_row, batch_row, batch_col, counts)


def _sc_gather(idx, table):
    D = table.shape[1]
    info = plsc.get_sparse_core_info()
    nc, ns = info.num_cores, info.num_subcores
    nw = nc * ns
    per_w = NEPAD // nw
    nch = per_w // KROWS

    nring = 4
    nrounds = nch // nring

    @functools.partial(
        pl.kernel,
        mesh=plsc.VectorSubcoreMesh(core_axis_name="c", subcore_axis_name="s"),
        out_type=jax.ShapeDtypeStruct((NEPAD, D), jnp.float32),
        scratch_types=[
            pltpu.VMEM((nring, KROWS), jnp.int32),
            pltpu.VMEM((nring, KROWS, D), jnp.float32),
            pltpu.SemaphoreType.DMA,
            pltpu.SemaphoreType.DMA,
            pltpu.SemaphoreType.DMA,
        ],
    )
    def gk(idx_hbm, tab_hbm, out_hbm, idx_v, rows_v, isem, gsem, wsem):
        wid = lax.axis_index("s") * nc + lax.axis_index("c")
        base = wid * per_w

        def round_body(r, carry):
            offs = [base + (r * nring + b) * KROWS for b in range(nring)]
            hi = [pltpu.async_copy(idx_hbm.at[pl.ds(offs[b], KROWS)],
                                   idx_v.at[b], isem) for b in range(nring)]
            hg = []
            for b in range(nring):
                hi[b].wait()
                hg.append(pltpu.async_copy(tab_hbm.at[idx_v.at[b]],
                                           rows_v.at[b], gsem))
            hw = []
            for b in range(nring):
                hg[b].wait()
                hw.append(pltpu.async_copy(rows_v.at[b],
                                           out_hbm.at[pl.ds(offs[b], KROWS)],
                                           wsem))
            for b in range(nring):
                hw[b].wait()
            return carry

        lax.fori_loop(0, nrounds, round_body, 0)

    return gk(idx, table)


def _mp_body(h_ref, ysrc_ref, d2_ref, w2_ref, bm_ref, wu_ref,
             bu_ref, wn_ref, bn_ref, hout_ref, ynext_ref):
    d2 = jnp.maximum(d2_ref[...], 0.0)
    d3 = jnp.sqrt(d2 + 1e-8).reshape(MROWS, KNN, 1)
    centers = lax.broadcasted_iota(jnp.int32, (1, 1, NG), 2).astype(jnp.float32) * 2.0
    rbf = jnp.exp(-_GAMMA * (d3 - centers) ** 2).reshape(MEDGE, NG)
    rw = jnp.dot(rbf, w2_ref[...], preferred_element_type=jnp.float32)
    m = jnp.maximum(ysrc_ref[...] + rw + bm_ref[...], 0.0)
    agg = jnp.sum(m.reshape(MROWS, KNN, H), axis=1)
    upd = jnp.maximum(
        jnp.dot(agg, wu_ref[...], preferred_element_type=jnp.float32)
        + bu_ref[...], 0.0)
    hout = h_ref[...] + upd
    hout_ref[...] = hout
    ynext_ref[...] = jnp.dot(hout, wn_ref[...],
                             preferred_element_type=jnp.float32) + bn_ref[...]


def _mp_layer(h, y_src, d2sel, W2, bm, Wu, bu, Wn, bn):
    return pl.pallas_call(
        _mp_body,
        grid=(MGRID,),
        in_specs=[
            pl.BlockSpec((MROWS, H), lambda i: (i, 0)),
            pl.BlockSpec((MEDGE, H), lambda i: (i, 0)),
            pl.BlockSpec((MROWS, KNN), lambda i: (i, 0)),
            pl.BlockSpec((NG, H), lambda i: (0, 0)),
            pl.BlockSpec((1, H), lambda i: (0, 0)),
            pl.BlockSpec((H, H), lambda i: (0, 0)),
            pl.BlockSpec((1, H), lambda i: (0, 0)),
            pl.BlockSpec((H, H), lambda i: (0, 0)),
            pl.BlockSpec((1, H), lambda i: (0, 0)),
        ],
        out_specs=(pl.BlockSpec((MROWS, H), lambda i: (i, 0)),
                   pl.BlockSpec((MROWS, H), lambda i: (i, 0))),
        out_shape=(jax.ShapeDtypeStruct((NPAD, H), jnp.float32),
                   jax.ShapeDtypeStruct((NPAD, H), jnp.float32)),
    )(h, y_src, d2sel, W2, bm, Wu, bu, Wn, bn)


def _y0_body(h_ref, w_ref, y_ref):
    y_ref[...] = jnp.dot(h_ref[...], w_ref[...],
                         preferred_element_type=jnp.float32)


def _y0(h, W1):
    return pl.pallas_call(
        _y0_body,
        grid=(MGRID,),
        in_specs=[pl.BlockSpec((MROWS, H), lambda i: (i, 0)),
                  pl.BlockSpec((H, H), lambda i: (0, 0))],
        out_specs=pl.BlockSpec((MROWS, H), lambda i: (i, 0)),
        out_shape=jax.ShapeDtypeStruct((NPAD, H), jnp.float32),
    )(h, W1)


def kernel(protein_pos, protein_atom_feature, ligand_pos, ligand_atom_feature,
           batch_protein, batch_ligand, Wp, bp, Wl, bl, W_msg, b_msg,
           W_upd, b_upd, Wf, bf):
    bpn = batch_protein.astype(jnp.int32)
    bln = batch_ligand.astype(jnp.int32)

    h_p, h_l = _embed(protein_atom_feature, Wp, bp, ligand_atom_feature, Wl, bl)

    dest_p = jnp.arange(NP_, dtype=jnp.int32) + jnp.searchsorted(
        bln, bpn, side="left").astype(jnp.int32)
    dest_l = jnp.searchsorted(bpn, bln, side="right").astype(jnp.int32) + \
        jnp.arange(NL, dtype=jnp.int32)
    dest = jnp.concatenate([dest_p, dest_l], 0)

    pos_all = jnp.concatenate([protein_pos, ligand_pos], 0)
    h_all = jnp.concatenate([h_p, h_l], 0)
    batch_all = jnp.concatenate([bpn, bln], 0)
    mask_all = jnp.concatenate([jnp.ones((NP_,), bool), jnp.zeros((NL,), bool)], 0)

    pos_ctx = jnp.zeros((N, 3), jnp.float32).at[dest].set(pos_all)
    h_ctx = jnp.zeros((N, H), jnp.float32).at[dest].set(h_all)
    batch_ctx = jnp.zeros((N,), jnp.int32).at[dest].set(batch_all)
    protein_mask = jnp.zeros((N,), bool).at[dest].set(mask_all)

    pos16 = jnp.zeros((NPAD, 16), jnp.float32).at[:N, :3].set(pos_ctx)
    pos8 = pos16[:, :8]
    posT = pos8.T.reshape(8, NPAD)
    sq = jnp.sum(pos_ctx * pos_ctx, axis=1)
    sq_row = jnp.full((1, NPAD), 3e9, jnp.float32).at[0, :N].set(sq)
    bpad = jnp.full((NPAD,), -1, jnp.int32).at[:N].set(batch_ctx)
    batch_row = bpad.reshape(1, NPAD)
    batch_col = bpad.reshape(NPAD, 1)
    hpad = jnp.zeros((NPAD, H), jnp.float32).at[:N].set(h_ctx)

    counts = jnp.zeros((B,), jnp.int32).at[batch_ctx].add(1).reshape(1, B)
    knn_idx, d2sel = _knn(pos8, posT, sq_row, batch_row, batch_col, counts)
    src = knn_idx.reshape(-1)

    h = hpad
    y = _y0(h, W_msg[0, :H, :])
    Wf_pad = jnp.zeros((H, H), jnp.float32).at[:, :1].set(Wf)
    bf_pad = jnp.zeros((1, H), jnp.float32).at[0, :1].set(bf)
    zrow = jnp.zeros((1, H), jnp.float32)
    for l in range(LAYERS):
        y_src = _sc_gather(src, y)
        if l + 1 < LAYERS:
            Wn, bn = W_msg[l + 1, :H, :], zrow
        else:
            Wn, bn = Wf_pad, bf_pad
        h, y = _mp_layer(h, y_src, d2sel,
                         W_msg[l, H:, :], b_msg[l].reshape(1, H),
                         W_upd[l], b_upd[l].reshape(1, H), Wn, bn)

    focal_pred = y[:N, :1]
    return focal_pred, protein_mask, h[:N]


# speedup vs baseline: 1.0180x; 1.0061x over previous
import functools

import jax
import jax.numpy as jnp
import numpy as np
from jax import lax
from jax.experimental import pallas as pl
from jax.experimental.pallas import tpu as pltpu
from jax.experimental.pallas import tpu_sc as plsc

H = 128
PD = 27
LD = 13
KNN = 16
NG = 16
LAYERS = 3
B = 256
NP_ = 8000
NL = 2000
N = NP_ + NL
NPAD = 10240
NEPAD = NPAD * KNN
KROWS = 128
KGRID = NPAD // KROWS
MROWS = 256
MEDGE = MROWS * KNN
MGRID = NPAD // MROWS

_GAMMA = 1.0 / (2.0 * (30.0 / NG) ** 2)


def _embed_body(fp_ref, wp_ref, bp_ref, fl_ref, wl_ref, bl_ref, hp_ref, hl_ref):
    hp_ref[...] = jnp.dot(fp_ref[...], wp_ref[...],
                          preferred_element_type=jnp.float32) + bp_ref[...]
    hl_ref[...] = jnp.dot(fl_ref[...], wl_ref[...],
                          preferred_element_type=jnp.float32) + bl_ref[...]


def _embed(feat_p, Wp, bp, feat_l, Wl, bl):
    return pl.pallas_call(
        _embed_body,
        out_shape=(jax.ShapeDtypeStruct((NP_, H), jnp.float32),
                   jax.ShapeDtypeStruct((NL, H), jnp.float32)),
    )(feat_p, Wp, bp.reshape(1, H), feat_l, Wl, bl.reshape(1, H))


WWIN = 512


def _knn_body(pos8_ref, posT_ref, sq_ref, brow_ref, bcol_ref, cnt_ref,
              idx_ref, d2_ref, sc_ref, sw_ref):
    pid = pl.program_id(0)
    pc = pos8_ref[...]
    rowsq = jnp.sum(pc * pc, axis=1, keepdims=True)
    grow = pid * KROWS + lax.broadcasted_iota(jnp.int32, (KROWS, 1), 0)
    bc = bcol_ref[...].reshape(KROWS, 1)
    br = brow_ref[...]

    bc_min = jnp.min(jnp.where(bc >= 0, bc, jnp.int32(2 ** 30)))
    bc_max = jnp.max(bc)
    col1 = lax.broadcasted_iota(jnp.int32, (1, NPAD), 1)
    lo = jnp.min(jnp.where(br == bc_min, col1, NPAD))
    hi = jnp.max(jnp.where(br == bc_max, col1 + 1, 0))
    lo_al = pl.multiple_of(jnp.clip((lo // 128) * 128, 0, NPAD - WWIN), 128)
    bidx = lax.broadcasted_iota(jnp.int32, (1, B), 1)
    inrange = (bidx >= bc_min) & (bidx <= bc_max)
    minseg = jnp.min(jnp.where(inrange, cnt_ref[...], jnp.int32(2 ** 30)))
    fast = (minseg >= KNN + 1) & (hi - lo_al <= WWIN)

    @pl.when(fast)
    def _fast():
        pcw = posT_ref[:, pl.ds(lo_al, WWIN)]
        d2 = rowsq + sq_ref[:, pl.ds(lo_al, WWIN)] - 2.0 * jnp.dot(
            pc, pcw, preferred_element_type=jnp.float32)
        colw = lo_al + lax.broadcasted_iota(jnp.int32, (KROWS, WWIN), 1)
        pen = jnp.where(brow_ref[:, pl.ds(lo_al, WWIN)] != bc, 1e9, 0.0)
        pen = pen + jnp.where(colw == grow, 1e9, 0.0)
        sw_ref[...] = d2 + pen
        for j in range(KNN):
            s = sw_ref[...]
            m = jnp.min(s, axis=1, keepdims=True)
            idx = jnp.min(jnp.where(s == m, colw, NPAD), axis=1, keepdims=True)
            idx_ref[:, j:j + 1] = idx
            d2_ref[:, j:j + 1] = m
            sw_ref[...] = jnp.where(colw == idx, jnp.inf, s)

    @pl.when(jnp.logical_not(fast))
    def _slow():
        dot = jnp.dot(pc, posT_ref[...], preferred_element_type=jnp.float32)
        d2 = rowsq + sq_ref[...] - 2.0 * dot
        col = lax.broadcasted_iota(jnp.int32, (KROWS, NPAD), 1)
        pen = jnp.where(br != bc, 1e9, 0.0)
        pen = pen + jnp.where(col == grow, 1e9, 0.0)
        sc_ref[...] = d2 + pen
        for j in range(KNN):
            s = sc_ref[...]
            m = jnp.min(s, axis=1, keepdims=True)
            idx = jnp.min(jnp.where(s == m, col, NPAD), axis=1, keepdims=True)
            idx_ref[:, j:j + 1] = idx
            psel = jnp.where(m < 5e8, 0.0, jnp.where(m < 1.5e9, 1e9, 2e9))
            d2_ref[:, j:j + 1] = m - psel
            sc_ref[...] = jnp.where(col == idx, jnp.inf, s)


def _knn(pos8, posT, sq_row, batch_row, batch_col, counts):
    return pl.pallas_call(
        _knn_body,
        grid=(KGRID,),
        in_specs=[
            pl.BlockSpec((KROWS, 8), lambda i: (i, 0)),
            pl.BlockSpec((8, NPAD), lambda i: (0, 0)),
            pl.BlockSpec((1, NPAD), lambda i: (0, 0)),
            pl.BlockSpec((1, NPAD), lambda i: (0, 0)),
            pl.BlockSpec((KROWS, 1), lambda i: (i, 0)),
            pl.BlockSpec((1, B), lambda i: (0, 0)),
        ],
        out_specs=(pl.BlockSpec((KROWS, KNN), lambda i: (i, 0)),
                   pl.BlockSpec((KROWS, KNN), lambda i: (i, 0))),
        out_shape=(jax.ShapeDtypeStruct((NPAD, KNN), jnp.int32),
                   jax.ShapeDtypeStruct((NPAD, KNN), jnp.float32)),
        scratch_shapes=[pltpu.VMEM((KROWS, NPAD), jnp.float32),
                        pltpu.VMEM((KROWS, WWIN), jnp.float32)],
    )(pos8, posT, sq_row, batch_row, batch_col, counts)


def _sc_gather(idx, table):
    D = table.shape[1]
    info = plsc.get_sparse_core_info()
    nc, ns = info.num_cores, info.num_subcores
    nw = nc * ns
    per_w = NEPAD // nw
    nch = per_w // KROWS

    nring = 5
    nrounds = nch // nring

    @functools.partial(
        pl.kernel,
        mesh=plsc.VectorSubcoreMesh(core_axis_name="c", subcore_axis_name="s"),
        out_type=jax.ShapeDtypeStruct((NEPAD, D), jnp.float32),
        scratch_types=[
            pltpu.VMEM((per_w,), jnp.int32),
            pltpu.VMEM((nring, KROWS, D), jnp.float32),
            pltpu.SemaphoreType.DMA,
            pltpu.SemaphoreType.DMA,
        ],
    )
    def gk(idx_hbm, tab_hbm, out_hbm, idx_v, rows_v, gsem, wsem):
        wid = lax.axis_index("s") * nc + lax.axis_index("c")
        base = wid * per_w
        pltpu.sync_copy(idx_hbm.at[pl.ds(base, per_w)], idx_v)

        def round_body(r, carry):
            offs = [(r * nring + b) * KROWS for b in range(nring)]
            hg = [pltpu.async_copy(
                tab_hbm.at[idx_v.at[pl.ds(offs[b], KROWS)]],
                rows_v.at[b], gsem) for b in range(nring)]
            hw = []
            for b in range(nring):
                hg[b].wait()
                hw.append(pltpu.async_copy(rows_v.at[b],
                                           out_hbm.at[pl.ds(base + offs[b],
                                                            KROWS)],
                                           wsem))
            for b in range(nring):
                hw[b].wait()
            return carry

        lax.fori_loop(0, nrounds, round_body, 0)

    return gk(idx, table)


def _mp_body(h_ref, ysrc_ref, d2_ref, w2_ref, bm_ref, wu_ref,
             bu_ref, wn_ref, bn_ref, hout_ref, ynext_ref):
    d2 = jnp.maximum(d2_ref[...], 0.0)
    d3 = jnp.sqrt(d2 + 1e-8).reshape(MROWS, KNN, 1)
    centers = lax.broadcasted_iota(jnp.int32, (1, 1, NG), 2).astype(jnp.float32) * 2.0
    rbf = jnp.exp(-_GAMMA * (d3 - centers) ** 2).reshape(MEDGE, NG)
    rw = jnp.dot(rbf, w2_ref[...], preferred_element_type=jnp.float32)
    m = jnp.maximum(ysrc_ref[...] + rw + bm_ref[...], 0.0)
    agg = jnp.sum(m.reshape(MROWS, KNN, H), axis=1)
    upd = jnp.maximum(
        jnp.dot(agg, wu_ref[...], preferred_element_type=jnp.float32)
        + bu_ref[...], 0.0)
    hout = h_ref[...] + upd
    hout_ref[...] = hout
    ynext_ref[...] = jnp.dot(hout, wn_ref[...],
                             preferred_element_type=jnp.float32) + bn_ref[...]


def _mp_layer(h, y_src, d2sel, W2, bm, Wu, bu, Wn, bn):
    return pl.pallas_call(
        _mp_body,
        grid=(MGRID,),
        in_specs=[
            pl.BlockSpec((MROWS, H), lambda i: (i, 0)),
            pl.BlockSpec((MEDGE, H), lambda i: (i, 0)),
            pl.BlockSpec((MROWS, KNN), lambda i: (i, 0)),
            pl.BlockSpec((NG, H), lambda i: (0, 0)),
            pl.BlockSpec((1, H), lambda i: (0, 0)),
            pl.BlockSpec((H, H), lambda i: (0, 0)),
            pl.BlockSpec((1, H), lambda i: (0, 0)),
            pl.BlockSpec((H, H), lambda i: (0, 0)),
            pl.BlockSpec((1, H), lambda i: (0, 0)),
        ],
        out_specs=(pl.BlockSpec((MROWS, H), lambda i: (i, 0)),
                   pl.BlockSpec((MROWS, H), lambda i: (i, 0))),
        out_shape=(jax.ShapeDtypeStruct((NPAD, H), jnp.float32),
                   jax.ShapeDtypeStruct((NPAD, H), jnp.float32)),
    )(h, y_src, d2sel, W2, bm, Wu, bu, Wn, bn)


def _y0_body(h_ref, w_ref, y_ref):
    y_ref[...] = jnp.dot(h_ref[...], w_ref[...],
                         preferred_element_type=jnp.float32)


def _y0(h, W1):
    return pl.pallas_call(
        _y0_body,
        grid=(MGRID,),
        in_specs=[pl.BlockSpec((MROWS, H), lambda i: (i, 0)),
                  pl.BlockSpec((H, H), lambda i: (0, 0))],
        out_specs=pl.BlockSpec((MROWS, H), lambda i: (i, 0)),
        out_shape=jax.ShapeDtypeStruct((NPAD, H), jnp.float32),
    )(h, W1)


def kernel(protein_pos, protein_atom_feature, ligand_pos, ligand_atom_feature,
           batch_protein, batch_ligand, Wp, bp, Wl, bl, W_msg, b_msg,
           W_upd, b_upd, Wf, bf):
    bpn = batch_protein.astype(jnp.int32)
    bln = batch_ligand.astype(jnp.int32)

    h_p, h_l = _embed(protein_atom_feature, Wp, bp, ligand_atom_feature, Wl, bl)

    dest_p = jnp.arange(NP_, dtype=jnp.int32) + jnp.searchsorted(
        bln, bpn, side="left").astype(jnp.int32)
    dest_l = jnp.searchsorted(bpn, bln, side="right").astype(jnp.int32) + \
        jnp.arange(NL, dtype=jnp.int32)
    dest = jnp.concatenate([dest_p, dest_l], 0)

    pos_all = jnp.concatenate([protein_pos, ligand_pos], 0)
    h_all = jnp.concatenate([h_p, h_l], 0)
    batch_all = jnp.concatenate([bpn, bln], 0)
    mask_all = jnp.concatenate([jnp.ones((NP_,), bool), jnp.zeros((NL,), bool)], 0)

    pos_ctx = jnp.zeros((N, 3), jnp.float32).at[dest].set(pos_all)
    h_ctx = jnp.zeros((N, H), jnp.float32).at[dest].set(h_all)
    batch_ctx = jnp.zeros((N,), jnp.int32).at[dest].set(batch_all)
    protein_mask = jnp.zeros((N,), bool).at[dest].set(mask_all)

    pos16 = jnp.zeros((NPAD, 16), jnp.float32).at[:N, :3].set(pos_ctx)
    pos8 = pos16[:, :8]
    posT = pos8.T.reshape(8, NPAD)
    sq = jnp.sum(pos_ctx * pos_ctx, axis=1)
    sq_row = jnp.full((1, NPAD), 3e9, jnp.float32).at[0, :N].set(sq)
    bpad = jnp.full((NPAD,), -1, jnp.int32).at[:N].set(batch_ctx)
    batch_row = bpad.reshape(1, NPAD)
    batch_col = bpad.reshape(NPAD, 1)
    hpad = jnp.zeros((NPAD, H), jnp.float32).at[:N].set(h_ctx)

    counts = jnp.zeros((B,), jnp.int32).at[batch_ctx].add(1).reshape(1, B)
    knn_idx, d2sel = _knn(pos8, posT, sq_row, batch_row, batch_col, counts)
    src = knn_idx.reshape(-1)

    h = hpad
    y = _y0(h, W_msg[0, :H, :])
    Wf_pad = jnp.zeros((H, H), jnp.float32).at[:, :1].set(Wf)
    bf_pad = jnp.zeros((1, H), jnp.float32).at[0, :1].set(bf)
    zrow = jnp.zeros((1, H), jnp.float32)
    for l in range(LAYERS):
        y_src = _sc_gather(src, y)
        if l + 1 < LAYERS:
            Wn, bn = W_msg[l + 1, :H, :], zrow
        else:
            Wn, bn = Wf_pad, bf_pad
        h, y = _mp_layer(h, y_src, d2sel,
                         W_msg[l, H:, :], b_msg[l].reshape(1, H),
                         W_upd[l], b_upd[l].reshape(1, H), Wn, bn)

    focal_pred = y[:N, :1]
    return focal_pred, protein_mask, h[:N]


# speedup vs baseline: 1.0229x; 1.0048x over previous
import functools

import jax
import jax.numpy as jnp
from jax import lax
from jax.experimental import pallas as pl
from jax.experimental.pallas import tpu as pltpu
from jax.experimental.pallas import tpu_sc as plsc

H = 128
PD = 27
LD = 13
KNN = 16
NG = 16
LAYERS = 3
B = 256
NP_ = 8000
NL = 2000
N = NP_ + NL
NPAD = 10240
NEPAD = NPAD * KNN
KROWS = 128
KGRID = NPAD // KROWS
MROWS = 256
MEDGE = MROWS * KNN
MGRID = NPAD // MROWS

_GAMMA = 1.0 / (2.0 * (30.0 / NG) ** 2)


def _embed_body(fp_ref, wp_ref, bp_ref, fl_ref, wl_ref, bl_ref, hp_ref, hl_ref):
    hp_ref[...] = jnp.dot(fp_ref[...], wp_ref[...],
                          preferred_element_type=jnp.float32) + bp_ref[...]
    hl_ref[...] = jnp.dot(fl_ref[...], wl_ref[...],
                          preferred_element_type=jnp.float32) + bl_ref[...]


def _embed(feat_p, Wp, bp, feat_l, Wl, bl):
    return pl.pallas_call(
        _embed_body,
        out_shape=(jax.ShapeDtypeStruct((NP_, H), jnp.float32),
                   jax.ShapeDtypeStruct((NL, H), jnp.float32)),
    )(feat_p, Wp, bp.reshape(1, H), feat_l, Wl, bl.reshape(1, H))


WWIN = 512


def _knn_body(pos8_ref, posT_ref, sq_ref, brow_ref, bcol_ref, cnt_ref,
              idx_ref, d2_ref, sc_ref, sw_ref):
    pid = pl.program_id(0)
    pc = pos8_ref[...]
    rowsq = jnp.sum(pc * pc, axis=1, keepdims=True)
    grow = pid * KROWS + lax.broadcasted_iota(jnp.int32, (KROWS, 1), 0)
    bc = bcol_ref[...].reshape(KROWS, 1)
    br = brow_ref[...]

    bc_min = jnp.min(jnp.where(bc >= 0, bc, jnp.int32(2 ** 30)))
    bc_max = jnp.max(bc)
    col1 = lax.broadcasted_iota(jnp.int32, (1, NPAD), 1)
    lo = jnp.min(jnp.where(br == bc_min, col1, NPAD))
    hi = jnp.max(jnp.where(br == bc_max, col1 + 1, 0))
    lo_al = pl.multiple_of(jnp.clip((lo // 128) * 128, 0, NPAD - WWIN), 128)
    bidx = lax.broadcasted_iota(jnp.int32, (1, B), 1)
    inrange = (bidx >= bc_min) & (bidx <= bc_max)
    minseg = jnp.min(jnp.where(inrange, cnt_ref[...], jnp.int32(2 ** 30)))
    fast = (minseg >= KNN + 1) & (hi - lo_al <= WWIN)

    @pl.when(fast)
    def _fast():
        pcw = posT_ref[:, pl.ds(lo_al, WWIN)]
        d2 = rowsq + sq_ref[:, pl.ds(lo_al, WWIN)] - 2.0 * jnp.dot(
            pc, pcw, preferred_element_type=jnp.float32)
        colw = lo_al + lax.broadcasted_iota(jnp.int32, (KROWS, WWIN), 1)
        pen = jnp.where(brow_ref[:, pl.ds(lo_al, WWIN)] != bc, 1e9, 0.0)
        pen = pen + jnp.where(colw == grow, 1e9, 0.0)
        sw_ref[...] = d2 + pen
        for j in range(KNN):
            s = sw_ref[...]
            m = jnp.min(s, axis=1, keepdims=True)
            idx = jnp.min(jnp.where(s == m, colw, NPAD), axis=1, keepdims=True)
            idx_ref[:, j:j + 1] = idx
            d2_ref[:, j:j + 1] = m
            sw_ref[...] = jnp.where(colw == idx, jnp.inf, s)

    @pl.when(jnp.logical_not(fast))
    def _slow():
        dot = jnp.dot(pc, posT_ref[...], preferred_element_type=jnp.float32)
        d2 = rowsq + sq_ref[...] - 2.0 * dot
        col = lax.broadcasted_iota(jnp.int32, (KROWS, NPAD), 1)
        pen = jnp.where(br != bc, 1e9, 0.0)
        pen = pen + jnp.where(col == grow, 1e9, 0.0)
        sc_ref[...] = d2 + pen
        for j in range(KNN):
            s = sc_ref[...]
            m = jnp.min(s, axis=1, keepdims=True)
            idx = jnp.min(jnp.where(s == m, col, NPAD), axis=1, keepdims=True)
            idx_ref[:, j:j + 1] = idx
            psel = jnp.where(m < 5e8, 0.0, jnp.where(m < 1.5e9, 1e9, 2e9))
            d2_ref[:, j:j + 1] = m - psel
            sc_ref[...] = jnp.where(col == idx, jnp.inf, s)


def _knn(pos8, posT, sq_row, batch_row, batch_col, counts):
    return pl.pallas_call(
        _knn_body,
        grid=(KGRID,),
        in_specs=[
            pl.BlockSpec((KROWS, 8), lambda i: (i, 0)),
            pl.BlockSpec((8, NPAD), lambda i: (0, 0)),
            pl.BlockSpec((1, NPAD), lambda i: (0, 0)),
            pl.BlockSpec((1, NPAD), lambda i: (0, 0)),
            pl.BlockSpec((KROWS, 1), lambda i: (i, 0)),
            pl.BlockSpec((1, B), lambda i: (0, 0)),
        ],
        out_specs=(pl.BlockSpec((KROWS, KNN), lambda i: (i, 0)),
                   pl.BlockSpec((KROWS, KNN), lambda i: (i, 0))),
        out_shape=(jax.ShapeDtypeStruct((NPAD, KNN), jnp.int32),
                   jax.ShapeDtypeStruct((NPAD, KNN), jnp.float32)),
        scratch_shapes=[pltpu.VMEM((KROWS, NPAD), jnp.float32),
                        pltpu.VMEM((KROWS, WWIN), jnp.float32)],
    )(pos8, posT, sq_row, batch_row, batch_col, counts)


def _sc_gather(idx, table):
    D = table.shape[1]
    info = plsc.get_sparse_core_info()
    nc, ns = info.num_cores, info.num_subcores
    nw = nc * ns
    per_w = NEPAD // nw
    nch = per_w // KROWS

    nring = 5
    nrounds = nch // nring

    @functools.partial(
        pl.kernel,
        mesh=plsc.VectorSubcoreMesh(core_axis_name="c", subcore_axis_name="s"),
        out_type=jax.ShapeDtypeStruct((NEPAD, D), jnp.float32),
        scratch_types=[
            pltpu.VMEM((per_w,), jnp.int32),
            pltpu.VMEM((nring, KROWS, D), jnp.float32),
            pltpu.SemaphoreType.DMA,
            pltpu.SemaphoreType.DMA,
        ],
    )
    def gk(idx_hbm, tab_hbm, out_hbm, idx_v, rows_v, gsem, wsem):
        wid = lax.axis_index("s") * nc + lax.axis_index("c")
        base = wid * per_w
        pltpu.sync_copy(idx_hbm.at[pl.ds(base, per_w)], idx_v)

        def round_body(r, carry):
            offs = [(r * nring + b) * KROWS for b in range(nring)]
            hg = [pltpu.async_copy(
                tab_hbm.at[idx_v.at[pl.ds(offs[b], KROWS)]],
                rows_v.at[b], gsem) for b in range(nring)]
            hw = []
            for b in range(nring):
                hg[b].wait()
                hw.append(pltpu.async_copy(rows_v.at[b],
                                           out_hbm.at[pl.ds(base + offs[b],
                                                            KROWS)],
                                           wsem))
            for b in range(nring):
                hw[b].wait()
            return carry

        lax.fori_loop(0, nrounds, round_body, 0)

    return gk(idx, table)


def _mp_body(h_ref, ysrc_ref, d2_ref, w2_ref, bm_ref, wu_ref,
             bu_ref, wn_ref, bn_ref, hout_ref, ynext_ref):
    d2 = jnp.maximum(d2_ref[...], 0.0)
    d3 = jnp.sqrt(d2 + 1e-8).reshape(MROWS, KNN, 1)
    centers = lax.broadcasted_iota(jnp.int32, (1, 1, NG), 2).astype(jnp.float32) * 2.0
    rbf = jnp.exp(-_GAMMA * (d3 - centers) ** 2).reshape(MEDGE, NG)
    rw = jnp.dot(rbf, w2_ref[...], preferred_element_type=jnp.float32)
    m = jnp.maximum(ysrc_ref[...] + rw + bm_ref[...], 0.0)
    agg = jnp.sum(m.reshape(MROWS, KNN, H), axis=1)
    upd = jnp.maximum(
        jnp.dot(agg, wu_ref[...], preferred_element_type=jnp.float32)
        + bu_ref[...], 0.0)
    hout = h_ref[...] + upd
    hout_ref[...] = hout
    ynext_ref[...] = jnp.dot(hout, wn_ref[...],
                             preferred_element_type=jnp.float32) + bn_ref[...]


def _mp_layer(h, y_src, d2sel, W2, bm, Wu, bu, Wn, bn):
    return pl.pallas_call(
        _mp_body,
        grid=(MGRID,),
        in_specs=[
            pl.BlockSpec((MROWS, H), lambda i: (i, 0)),
            pl.BlockSpec((MEDGE, H), lambda i: (i, 0)),
            pl.BlockSpec((MROWS, KNN), lambda i: (i, 0)),
            pl.BlockSpec((NG, H), lambda i: (0, 0)),
            pl.BlockSpec((1, H), lambda i: (0, 0)),
            pl.BlockSpec((H, H), lambda i: (0, 0)),
            pl.BlockSpec((1, H), lambda i: (0, 0)),
            pl.BlockSpec((H, H), lambda i: (0, 0)),
            pl.BlockSpec((1, H), lambda i: (0, 0)),
        ],
        out_specs=(pl.BlockSpec((MROWS, H), lambda i: (i, 0)),
                   pl.BlockSpec((MROWS, H), lambda i: (i, 0))),
        out_shape=(jax.ShapeDtypeStruct((NPAD, H), jnp.float32),
                   jax.ShapeDtypeStruct((NPAD, H), jnp.float32)),
    )(h, y_src, d2sel, W2, bm, Wu, bu, Wn, bn)


def _y0_body(h_ref, w_ref, y_ref):
    y_ref[...] = jnp.dot(h_ref[...], w_ref[...],
                         preferred_element_type=jnp.float32)


def _y0(h, W1):
    return pl.pallas_call(
        _y0_body,
        grid=(MGRID,),
        in_specs=[pl.BlockSpec((MROWS, H), lambda i: (i, 0)),
                  pl.BlockSpec((H, H), lambda i: (0, 0))],
        out_specs=pl.BlockSpec((MROWS, H), lambda i: (i, 0)),
        out_shape=jax.ShapeDtypeStruct((NPAD, H), jnp.float32),
    )(h, W1)


def kernel(protein_pos, protein_atom_feature, ligand_pos, ligand_atom_feature,
           batch_protein, batch_ligand, Wp, bp, Wl, bl, W_msg, b_msg,
           W_upd, b_upd, Wf, bf):
    bpn = batch_protein.astype(jnp.int32)
    bln = batch_ligand.astype(jnp.int32)

    h_p, h_l = _embed(protein_atom_feature, Wp, bp, ligand_atom_feature, Wl, bl)

    dest_p = jnp.arange(NP_, dtype=jnp.int32) + jnp.searchsorted(
        bln, bpn, side="left").astype(jnp.int32)
    dest_l = jnp.searchsorted(bpn, bln, side="right").astype(jnp.int32) + \
        jnp.arange(NL, dtype=jnp.int32)
    dest = jnp.concatenate([dest_p, dest_l], 0)

    pos_all = jnp.concatenate([protein_pos, ligand_pos], 0)
    h_all = jnp.concatenate([h_p, h_l], 0)
    batch_all = jnp.concatenate([bpn, bln], 0)
    mask_all = jnp.concatenate([jnp.ones((NP_,), bool), jnp.zeros((NL,), bool)], 0)

    pos_ctx = jnp.zeros((N, 3), jnp.float32).at[dest].set(pos_all)
    h_ctx = jnp.zeros((N, H), jnp.float32).at[dest].set(h_all)
    batch_ctx = jnp.zeros((N,), jnp.int32).at[dest].set(batch_all)
    protein_mask = jnp.zeros((N,), bool).at[dest].set(mask_all)

    pos8 = jnp.zeros((NPAD, 8), jnp.float32).at[:N, :3].set(pos_ctx)
    posT = pos8.T.reshape(8, NPAD)
    sq = jnp.sum(pos_ctx * pos_ctx, axis=1)
    sq_row = jnp.full((1, NPAD), 3e9, jnp.float32).at[0, :N].set(sq)
    bpad = jnp.full((NPAD,), -1, jnp.int32).at[:N].set(batch_ctx)
    batch_row = bpad.reshape(1, NPAD)
    batch_col = bpad.reshape(NPAD, 1)
    hpad = jnp.zeros((NPAD, H), jnp.float32).at[:N].set(h_ctx)

    counts = jnp.zeros((B,), jnp.int32).at[batch_ctx].add(1).reshape(1, B)
    knn_idx, d2sel = _knn(pos8, posT, sq_row, batch_row, batch_col, counts)
    src = knn_idx.reshape(-1)

    h = hpad
    y = _y0(h, W_msg[0, :H, :])
    Wf_pad = jnp.zeros((H, H), jnp.float32).at[:, :1].set(Wf)
    bf_pad = jnp.zeros((1, H), jnp.float32).at[0, :1].set(bf)
    zrow = jnp.zeros((1, H), jnp.float32)
    for l in range(LAYERS):
        y_src = _sc_gather(src, y)
        if l + 1 < LAYERS:
            Wn, bn = W_msg[l + 1, :H, :], zrow
        else:
            Wn, bn = Wf_pad, bf_pad
        h, y = _mp_layer(h, y_src, d2sel,
                         W_msg[l, H:, :], b_msg[l].reshape(1, H),
                         W_upd[l], b_upd[l].reshape(1, H), Wn, bn)

    focal_pred = y[:N, :1]
    return focal_pred, protein_mask, h[:N]
